# baseline (speedup 1.0000x reference)
import functools

import jax
import jax.numpy as jnp
from jax import lax
from jax.experimental import pallas as pl
from jax.experimental.pallas import tpu as pltpu
from jax.experimental.pallas import tpu_sc as plsc

NC = 2
NS = 16
NW = NC * NS
D = 64
W = D // 2
C = 800
L = 16
G = C // L


@functools.partial(jax.jit, static_argnums=(2,))
def _gather_rows(idx, packed, B):
    b_per_w = B // NW
    n_chunks = b_per_w // C
    mesh = plsc.VectorSubcoreMesh(
        core_axis_name="c", subcore_axis_name="s",
        num_cores=NC, num_subcores=NS)

    @functools.partial(
        pl.kernel,
        out_type=jax.ShapeDtypeStruct((B, W), jnp.int32),
        mesh=mesh,
        scratch_types=[
            pltpu.VMEM((n_chunks, C), jnp.int32),
            pltpu.VMEM((C, W), jnp.int32),
            pltpu.VMEM((C, W), jnp.int32),
            pltpu.SemaphoreType.DMA,
            pltpu.SemaphoreType.DMA,
        ],
        compiler_params=pltpu.CompilerParams(use_tc_tiling_on_sc=False),
    )
    def k(idx_hbm, tab_hbm, out_hbm, idx_v, rows0, rows1, sg0, sg1):
        wid = lax.axis_index("s") * NC + lax.axis_index("c")
        wc0 = wid * n_chunks
        rows = (rows0, rows1)
        sg = (sg0, sg1)

        pltpu.sync_copy(idx_hbm.at[pl.ds(wc0, n_chunks)], idx_v)

        def gather_start(g, b):
            for q in range(G):
                iv = idx_v[g, pl.ds(L * q, L)]
                pltpu.async_copy(
                    tab_hbm.at[iv], rows[b].at[pl.ds(L * q, L)], sg[b])

        def gather_wait(g, b):
            pltpu.make_async_copy(
                tab_hbm.at[pl.ds(0, C)], rows[b], sg[b]).wait()

        gather_start(0, 0)
        gather_start(1, 1)

        def block(i, carry):
            t = 2 * i
            for b in (0, 1):
                g = t + b
                gather_wait(g - 2, b)
                gather_start(g, b)
            return carry

        lax.fori_loop(1, n_chunks // 2, block, 0)

        gather_wait(n_chunks - 2, 0)
        gather_wait(n_chunks - 1, 1)

    return k(idx, packed)


def kernel(edge_type, position_embedding):
    s0, s1 = edge_type.shape
    B = s0 * s1
    idx = edge_type.reshape(B // C, C).astype(jnp.int32)
    packed = lax.bitcast_convert_type(
        position_embedding.astype(jnp.bfloat16).reshape(-1, W, 2), jnp.int32)
    return _gather_rows(idx, packed, B)


# speedup vs baseline: 1.0153x; 1.0153x over previous
import functools

import jax
import jax.numpy as jnp
from jax import lax
from jax.experimental import pallas as pl
from jax.experimental.pallas import tpu as pltpu
from jax.experimental.pallas import tpu_sc as plsc

NC = 2
NS = 16
NW = NC * NS
D = 64
C = 512
NBUF = 3


@functools.partial(jax.jit, static_argnums=(2,))
def _gather_rows(idx, table, B):
    b_per_w = B // NW
    n_chunks = b_per_w // C
    assert n_chunks > NBUF
    mesh = plsc.VectorSubcoreMesh(
        core_axis_name="c", subcore_axis_name="s",
        num_cores=NC, num_subcores=NS)

    @functools.partial(
        pl.kernel,
        out_type=jax.ShapeDtypeStruct((B, D), jnp.float32),
        mesh=mesh,
        scratch_types=[
            pltpu.VMEM((n_chunks, C), jnp.int32),
        ] + [pltpu.VMEM((C, D), jnp.float32)] * NBUF
          + [pltpu.SemaphoreType.DMA] * (2 * NBUF),
        compiler_params=pltpu.CompilerParams(use_tc_tiling_on_sc=False),
    )
    def k(idx_hbm, table_hbm, out_hbm, idx_v, *bufs):
        rows = bufs[:NBUF]
        sg = bufs[NBUF:2 * NBUF]
        so = bufs[2 * NBUF:]
        wid = lax.axis_index("s") * NC + lax.axis_index("c")
        wc0 = wid * n_chunks

        pltpu.sync_copy(idx_hbm.at[pl.ds(wc0, n_chunks)], idx_v)

        def gather_start(g, b):
            pltpu.async_copy(table_hbm.at[idx_v.at[g]], rows[b], sg[b])

        def gather_wait(g, b):
            pltpu.make_async_copy(table_hbm.at[idx_v.at[g]], rows[b], sg[b]).wait()

        def out_start(g, b):
            base = (wc0 + g) * C
            pltpu.async_copy(rows[b], out_hbm.at[pl.ds(base, C)], so[b])

        def out_wait(g, b):
            base = (wc0 + g) * C
            pltpu.make_async_copy(rows[b], out_hbm.at[pl.ds(base, C)], so[b]).wait()

        for g in range(n_chunks):
            b = g % NBUF
            if g >= NBUF:
                out_wait(g - NBUF, b)
            gather_start(g, b)
            if g >= 1:
                gather_wait(g - 1, (g - 1) % NBUF)
                out_start(g - 1, (g - 1) % NBUF)

        gl = n_chunks - 1
        gather_wait(gl, gl % NBUF)
        out_start(gl, gl % NBUF)
        for g in range(n_chunks - NBUF, n_chunks):
            out_wait(g, g % NBUF)

    return k(idx, table)


def kernel(edge_type, position_embedding):
    s0, s1 = edge_type.shape
    B = s0 * s1
    idx = edge_type.reshape(B // C, C).astype(jnp.int32)
    out = _gather_rows(idx, position_embedding, B)
    return out.reshape(s0, s1, D)


# speedup vs baseline: 1.0204x; 1.0051x over previous
import functools

import jax
import jax.numpy as jnp
from jax import lax
from jax.experimental import pallas as pl
from jax.experimental.pallas import tpu as pltpu
from jax.experimental.pallas import tpu_sc as plsc

NC = 2
NS = 16
NW = NC * NS
D = 64
C = 800


@functools.partial(jax.jit, static_argnums=(2,))
def _gather_rows(idx, table, B):
    b_per_w = B // NW
    n_chunks = b_per_w // C
    assert n_chunks % 2 == 0 and n_chunks >= 4
    mesh = plsc.VectorSubcoreMesh(
        core_axis_name="c", subcore_axis_name="s",
        num_cores=NC, num_subcores=NS)

    @functools.partial(
        pl.kernel,
        out_type=jax.ShapeDtypeStruct((B, D), jnp.float32),
        mesh=mesh,
        scratch_types=[
            pltpu.VMEM((n_chunks, C), jnp.int32),
            pltpu.VMEM((C, D), jnp.float32),
            pltpu.VMEM((C, D), jnp.float32),
            pltpu.SemaphoreType.DMA,
            pltpu.SemaphoreType.DMA,
            pltpu.SemaphoreType.DMA,
            pltpu.SemaphoreType.DMA,
        ],
        compiler_params=pltpu.CompilerParams(use_tc_tiling_on_sc=False),
    )
    def k(idx_hbm, table_hbm, out_hbm, idx_v, rows0, rows1, sg0, sg1, so0, so1):
        wid = lax.axis_index("s") * NC + lax.axis_index("c")
        wc0 = wid * n_chunks
        rows = (rows0, rows1)
        sg = (sg0, sg1)
        so = (so0, so1)

        pltpu.sync_copy(idx_hbm.at[pl.ds(wc0, n_chunks)], idx_v)

        def gather_start(g, b):
            pltpu.async_copy(table_hbm.at[idx_v.at[g]], rows[b], sg[b])

        def out_start(g, b):
            base = (wc0 + g) * C
            pltpu.async_copy(rows[b], out_hbm.at[pl.ds(base, C)], so[b])

        def gather_wait(g, b):
            pltpu.make_async_copy(table_hbm.at[idx_v.at[g]], rows[b], sg[b]).wait()

        def out_wait(g, b):
            base = (wc0 + g) * C
            pltpu.make_async_copy(rows[b], out_hbm.at[pl.ds(base, C)], so[b]).wait()

        gather_start(0, 0)
        gather_start(1, 1)
        gather_wait(0, 0)
        out_start(0, 0)

        def block(i, carry):
            t = 2 * i
            for b in (0, 1):
                g = t + b
                out_wait(g - 2, b)
                gather_start(g, b)
                gather_wait(g - 1, 1 - b)
                out_start(g - 1, 1 - b)
            return carry

        lax.fori_loop(1, n_chunks // 2, block, 0)

        gl = n_chunks - 1
        gather_wait(gl, gl % 2)
        out_start(gl, gl % 2)
        out_wait(gl - 1, (gl - 1) % 2)
        out_wait(gl, gl % 2)

    return k(idx, table)


def kernel(edge_type, position_embedding):
    s0, s1 = edge_type.shape
    B = s0 * s1
    idx = edge_type.reshape(B // C, C).astype(jnp.int32)
    out = _gather_rows(idx, position_embedding, B)
    return out.reshape(s0, s1, D)
